# Initial kernel scaffold; baseline (speedup 1.0000x reference)
import jax
import jax.numpy as jnp
from jax import lax
from jax.experimental import pallas as pl
from jax.experimental.pallas import tpu as pltpu
from jax.experimental.pallas import tpu_sc as plsc

N = 100000
NPAD = 102400
G = 128
GP = 136
CH = 128
BK = 16
NSUB = 16
NCORE = 2
NW = NCORE * NSUB
ZR = 800
NRW = NPAD // NSUB


def _mesh():
    return plsc.VectorSubcoreMesh(core_axis_name="c", subcore_axis_name="s")


def _deg_kernel(ech):
    wch = ech // NW
    nblk = wch // BK

    def body(dst_ref, zrow_ref, out_ref, acc, didx, ones, ssem):
        c = lax.axis_index("c")
        s = lax.axis_index("s")
        for i in range(CH // 16):
            ones[pl.ds(i * 16, 16)] = jnp.ones((16,), jnp.float32)

        def zloop(k, _):
            pltpu.sync_copy(zrow_ref, acc.at[pl.ds(s * NRW + k * ZR, ZR)])
            return 0
        lax.fori_loop(0, NRW // ZR, zloop, 0)
        plsc.subcore_barrier()

        base = (c * NSUB + s) * wch

        def blk(b, _):
            cr = base + b * BK
            pltpu.sync_copy(dst_ref.at[pl.ds(cr, BK)], didx)
            ds2 = [pltpu.async_copy(ones, acc.at[didx.at[j]], ssem, add=True)
                   for j in range(BK)]
            for d in ds2:
                d.wait()
            return 0
        lax.fori_loop(0, nblk, blk, 0)
        plsc.subcore_barrier()
        pltpu.sync_copy(acc.at[pl.ds(s * NRW, NRW)],
                        out_ref.at[c].at[pl.ds(s * NRW, NRW)])

    return pl.kernel(
        body,
        out_type=jax.ShapeDtypeStruct((NCORE, NPAD), jnp.float32),
        mesh=_mesh(),
        scratch_types=[
            pltpu.VMEM_SHARED((NPAD,), jnp.float32),
            pltpu.VMEM((BK, CH), jnp.int32),
            pltpu.VMEM((CH,), jnp.float32),
            pltpu.SemaphoreType.DMA,
        ],
    )


def _agg_kernel(F, edge_split, ech):
    wch = ech // NW if edge_split else ech // NSUB
    nblk = wch // BK

    def body(g_ref, src_ref, dst_ref, zrow_ref, out_ref,
             acc, sidx, didx, rows, gsem, ssem):
        c = lax.axis_index("c")
        s = lax.axis_index("s")

        def zloop(k, _):
            pltpu.sync_copy(zrow_ref, acc.at[pl.ds(s * NRW + k * ZR, ZR)])
            return 0
        lax.fori_loop(0, NRW // ZR, zloop, 0)
        plsc.subcore_barrier()

        if edge_split:
            base = (c * NSUB + s) * wch
            table = g_ref
        else:
            base = s * wch
            table = g_ref.at[c]

        def blk(b, _):
            cr = base + b * BK
            pltpu.sync_copy(src_ref.at[pl.ds(cr, BK)], sidx)
            pltpu.sync_copy(dst_ref.at[pl.ds(cr, BK)], didx)
            ds1 = [pltpu.async_copy(table.at[sidx.at[j]], rows.at[j], gsem)
                   for j in range(BK)]
            for d in ds1:
                d.wait()
            ds2 = [pltpu.async_copy(rows.at[j], acc.at[didx.at[j]], ssem,
                                    add=True)
                   for j in range(BK)]
            for d in ds2:
                d.wait()
            return 0
        lax.fori_loop(0, nblk, blk, 0)
        plsc.subcore_barrier()
        pltpu.sync_copy(acc.at[pl.ds(s * NRW, NRW)],
                        out_ref.at[c].at[pl.ds(s * NRW, NRW)])

    return pl.kernel(
        body,
        out_type=jax.ShapeDtypeStruct((NCORE, NPAD, F), jnp.float32),
        mesh=_mesh(),
        scratch_types=[
            pltpu.VMEM_SHARED((NPAD, F), jnp.float32),
            pltpu.VMEM((BK, CH), jnp.int32),
            pltpu.VMEM((BK, CH), jnp.int32),
            pltpu.VMEM((BK, CH, F), jnp.float32),
            pltpu.SemaphoreType.DMA,
            pltpu.SemaphoreType.DMA,
        ],
    )


def _pool_kernel():
    nchb = NPAD // CH
    wcr = nchb // NW

    def body(h_ref, b2d_ref, zp_ref, zc_ref, psum_ref, pcnt_ref,
             accp, accc, bidx, rows, ones, ssem):
        c = lax.axis_index("c")
        s = lax.axis_index("s")
        for i in range(CH // 16):
            ones[pl.ds(i * 16, 16)] = jnp.ones((16,), jnp.float32)

        @pl.when(s == 0)
        def _():
            pltpu.sync_copy(zp_ref, accp)
            pltpu.sync_copy(zc_ref, accc)
        plsc.subcore_barrier()

        base = (c * NSUB + s) * wcr
        pltpu.sync_copy(b2d_ref.at[pl.ds(base, wcr)], bidx)

        def blk(b, _):
            pltpu.sync_copy(h_ref.at[pl.ds((base + b) * CH, CH)], rows)
            pltpu.sync_copy(rows, accp.at[bidx.at[b]], add=True)
            pltpu.async_copy(ones, accc.at[bidx.at[b]], ssem, add=True).wait()
            return 0
        lax.fori_loop(0, wcr, blk, 0)
        plsc.subcore_barrier()

        @pl.when(s == 0)
        def _():
            pltpu.sync_copy(accp, psum_ref.at[c])
            pltpu.sync_copy(accc, pcnt_ref.at[c])

    return pl.kernel(
        body,
        out_type=(jax.ShapeDtypeStruct((NCORE, GP, 32), jnp.float32),
                  jax.ShapeDtypeStruct((NCORE, GP), jnp.float32)),
        mesh=_mesh(),
        scratch_types=[
            pltpu.VMEM_SHARED((GP, 32), jnp.float32),
            pltpu.VMEM_SHARED((GP,), jnp.float32),
            pltpu.VMEM((25, CH), jnp.int32),
            pltpu.VMEM((CH, 32), jnp.float32),
            pltpu.VMEM((CH,), jnp.float32),
            pltpu.SemaphoreType.DMA,
        ],
    )


RB = 2048
NBLK_TC = NPAD // RB


def _b1_body(degp, x, w1, g1, dinv):
    deg = degp[0] + degp[1] + 1.0
    di = lax.rsqrt(deg)
    h = jnp.dot(x[...], w1[...], preferred_element_type=jnp.float32)
    g1[...] = h * di[:, None]
    dinv[...] = di[:, None]


def _tc_b1(x_pad, degp, W1):
    return pl.pallas_call(
        _b1_body,
        grid=(NBLK_TC,),
        in_specs=[
            pl.BlockSpec((NCORE, RB), lambda i: (0, i)),
            pl.BlockSpec((RB, 9), lambda i: (i, 0)),
            pl.BlockSpec((9, 8), lambda i: (0, 0)),
        ],
        out_specs=[
            pl.BlockSpec((RB, 8), lambda i: (i, 0)),
            pl.BlockSpec((RB, 1), lambda i: (i, 0)),
        ],
        out_shape=[
            jax.ShapeDtypeStruct((NPAD, 8), jnp.float32),
            jax.ShapeDtypeStruct((NPAD, 1), jnp.float32),
        ],
    )(degp, x_pad, W1)


def _mid_body(accp, g, dinv, w, b, gout):
    di = dinv[...]
    t = di * (accp[0] + accp[1] + g[...]) + b[...]
    t = jnp.maximum(t, 0.0)
    h = jnp.dot(t, w[...], preferred_element_type=jnp.float32)
    gout[...] = h * di


def _tc_mid(accp, g, dinv, W, b, fin, fout):
    return pl.pallas_call(
        _mid_body,
        grid=(NBLK_TC,),
        in_specs=[
            pl.BlockSpec((NCORE, RB, fin), lambda i: (0, i, 0)),
            pl.BlockSpec((RB, fin), lambda i: (i, 0)),
            pl.BlockSpec((RB, 1), lambda i: (i, 0)),
            pl.BlockSpec((fin, fout), lambda i: (0, 0)),
            pl.BlockSpec((1, fin), lambda i: (0, 0)),
        ],
        out_specs=pl.BlockSpec((RB, fout), lambda i: (i, 0)),
        out_shape=jax.ShapeDtypeStruct((NPAD, fout), jnp.float32),
    )(accp, g, dinv, W, b)


def _b3_body(accp, g, dinv, w, b, gout):
    di = dinv[...]
    t = di * (accp[0] + accp[1] + g[...]) + b[...]
    t = jnp.maximum(t, 0.0)
    h = jnp.dot(t, w[...], preferred_element_type=jnp.float32)
    h = h * di
    gout[0] = h[:, :16]
    gout[1] = h[:, 16:]


def _tc_b3(accp, g2, dinv, W3, b2):
    return pl.pallas_call(
        _b3_body,
        grid=(NBLK_TC,),
        in_specs=[
            pl.BlockSpec((NCORE, RB, 16), lambda i: (0, i, 0)),
            pl.BlockSpec((RB, 16), lambda i: (i, 0)),
            pl.BlockSpec((RB, 1), lambda i: (i, 0)),
            pl.BlockSpec((16, 32), lambda i: (0, 0)),
            pl.BlockSpec((1, 16), lambda i: (0, 0)),
        ],
        out_specs=pl.BlockSpec((NCORE, RB, 16), lambda i: (0, i, 0)),
        out_shape=jax.ShapeDtypeStruct((NCORE, NPAD, 16), jnp.float32),
    )(accp, g2, dinv, W3, b2)


def _b4_body(acc3, g3, dinv, b, out):
    di = dinv[...]
    lo = di * (acc3[0] + g3[0])
    hi = di * (acc3[1] + g3[1])
    out[...] = jnp.concatenate([lo, hi], axis=1) + b[...]


def _tc_b4(acc3, g3, dinv, b3):
    return pl.pallas_call(
        _b4_body,
        grid=(NBLK_TC,),
        in_specs=[
            pl.BlockSpec((NCORE, RB, 16), lambda i: (0, i, 0)),
            pl.BlockSpec((NCORE, RB, 16), lambda i: (0, i, 0)),
            pl.BlockSpec((RB, 1), lambda i: (i, 0)),
            pl.BlockSpec((1, 32), lambda i: (0, 0)),
        ],
        out_specs=pl.BlockSpec((RB, 32), lambda i: (i, 0)),
        out_shape=jax.ShapeDtypeStruct((NPAD, 32), jnp.float32),
    )(acc3, g3, dinv, b3)


def _head_body(psum, pcnt, wl, bl, out):
    pooled = psum[0] + psum[1]
    cnt = pcnt[0] + pcnt[1]
    mean = pooled / jnp.maximum(cnt, 1.0)[:, None]
    r = jnp.dot(mean, wl[...], preferred_element_type=jnp.float32) + bl[...]
    out[...] = r[:G]


def _tc_head(psum, pcnt, Wl, bl):
    return pl.pallas_call(
        _head_body,
        out_shape=jax.ShapeDtypeStruct((G, 2), jnp.float32),
    )(psum, pcnt, Wl, bl)


def kernel(x, edge_index, batch, W1, b1, W2, b2, W3, b3, Wl, bl):
    E = edge_index.shape[1]
    wch = -(-E // (NW * CH * BK)) * BK
    ech = wch * NW
    e_pad = ech * CH

    src = jnp.concatenate(
        [edge_index[0], jnp.full((e_pad - E,), N, jnp.int32)]).reshape(ech, CH)
    dst = jnp.concatenate(
        [edge_index[1], jnp.full((e_pad - E,), N, jnp.int32)]).reshape(ech, CH)
    x_pad = jnp.concatenate(
        [x, jnp.zeros((NPAD - N, 9), jnp.float32)], axis=0)
    b2d = jnp.concatenate(
        [batch, jnp.full((NPAD - N,), G, jnp.int32)]).reshape(NPAD // CH, CH)

    zrow1 = jnp.zeros((ZR,), jnp.float32)
    zrow8 = jnp.zeros((ZR, 8), jnp.float32)
    zrow16 = jnp.zeros((ZR, 16), jnp.float32)
    zp = jnp.zeros((GP, 32), jnp.float32)
    zc = jnp.zeros((GP,), jnp.float32)

    degp = _deg_kernel(ech)(dst, zrow1)
    g1, dinv = _tc_b1(x_pad, degp, W1)
    acc1 = _agg_kernel(8, True, ech)(g1, src, dst, zrow8)
    g2 = _tc_mid(acc1, g1, dinv, W2, b1.reshape(1, 8), 8, 16)
    acc2 = _agg_kernel(16, True, ech)(g2, src, dst, zrow16)
    g3 = _tc_b3(acc2, g2, dinv, W3, b2.reshape(1, 16))
    acc3 = _agg_kernel(16, False, ech)(g3, src, dst, zrow16)
    out3 = _tc_b4(acc3, g3, dinv, b3.reshape(1, 32))
    psum, pcnt = _pool_kernel()(out3, b2d, zp, zc)
    return _tc_head(psum, pcnt, Wl, bl.reshape(1, 2))


# speedup vs baseline: 44.1477x; 44.1477x over previous
import jax
import jax.numpy as jnp
from jax import lax
from jax.experimental import pallas as pl
from jax.experimental.pallas import tpu as pltpu
from jax.experimental.pallas import tpu_sc as plsc

N = 100000
NPAD = 102400
G = 128
GP = 144
CH = 128
BK = 8
NSUB = 16
NCORE = 2
NW = NCORE * NSUB
ZR = 200
NRW = NPAD // NSUB
F = 16


def _mesh():
    return plsc.VectorSubcoreMesh(core_axis_name="c", subcore_axis_name="s")


_SC_PARAMS = pltpu.CompilerParams(use_tc_tiling_on_sc=False)


def _fill(ref, nrows, ncols, value):
    def row(i, _):
        for j in range(ncols // 16):
            ref[i, pl.ds(j * 16, 16)] = jnp.full((16,), value, jnp.float32)
        return 0
    lax.fori_loop(0, nrows, row, 0)


def _zero_acc(acc, zbuf, s):
    _fill(zbuf, ZR, F, 0.0)

    def zloop(k, _):
        pltpu.sync_copy(zbuf, acc.at[pl.ds(s * NRW + k * ZR, ZR)])
        return 0
    lax.fori_loop(0, NRW // ZR, zloop, 0)


def _deg_kernel(ech):
    wch = ech // NW
    nblk = wch // BK

    def body(dst_ref, out_ref, acc, didx, ones, zbuf, ssem):
        c = lax.axis_index("c")
        s = lax.axis_index("s")
        _fill(ones, CH, F, 1.0)
        _zero_acc(acc, zbuf, s)
        plsc.subcore_barrier()

        base = (c * NSUB + s) * wch

        def blk(b, _):
            cr = base + b * BK
            pltpu.sync_copy(dst_ref.at[pl.ds(cr, BK)], didx)
            ds2 = [pltpu.async_copy(ones, acc.at[didx.at[j]], ssem, add=True)
                   for j in range(BK)]
            for d in ds2:
                d.wait()
            return 0
        lax.fori_loop(0, nblk, blk, 0)
        plsc.subcore_barrier()
        pltpu.sync_copy(acc.at[pl.ds(s * NRW, NRW)],
                        out_ref.at[c].at[pl.ds(s * NRW, NRW)])

    return pl.kernel(
        body,
        out_type=jax.ShapeDtypeStruct((NCORE, NPAD, F), jnp.float32),
        mesh=_mesh(),
        compiler_params=_SC_PARAMS,
        scratch_types=[
            pltpu.VMEM_SHARED((NPAD, F), jnp.float32),
            pltpu.VMEM((BK, CH), jnp.int32),
            pltpu.VMEM((CH, F), jnp.float32),
            pltpu.VMEM((ZR, F), jnp.float32),
            pltpu.SemaphoreType.DMA,
        ],
    )


def _agg_kernel(edge_split, ech):
    wch = ech // NW if edge_split else ech // NSUB
    nblk = wch // BK

    def body(g_ref, src_ref, dst_ref, out_ref,
             acc, sidx, didx, rows, zbuf, gsem, ssem):
        c = lax.axis_index("c")
        s = lax.axis_index("s")
        _zero_acc(acc, zbuf, s)
        plsc.subcore_barrier()

        if edge_split:
            base = (c * NSUB + s) * wch
            table = g_ref
        else:
            base = s * wch
            table = g_ref.at[c]

        def blk(b, _):
            cr = base + b * BK
            pltpu.sync_copy(src_ref.at[pl.ds(cr, BK)], sidx)
            pltpu.sync_copy(dst_ref.at[pl.ds(cr, BK)], didx)
            ds1 = [pltpu.async_copy(table.at[sidx.at[j]], rows.at[j], gsem)
                   for j in range(BK)]
            for d in ds1:
                d.wait()
            ds2 = [pltpu.async_copy(rows.at[j], acc.at[didx.at[j]], ssem,
                                    add=True)
                   for j in range(BK)]
            for d in ds2:
                d.wait()
            return 0
        lax.fori_loop(0, nblk, blk, 0)
        plsc.subcore_barrier()
        pltpu.sync_copy(acc.at[pl.ds(s * NRW, NRW)],
                        out_ref.at[c].at[pl.ds(s * NRW, NRW)])

    return pl.kernel(
        body,
        out_type=jax.ShapeDtypeStruct((NCORE, NPAD, F), jnp.float32),
        mesh=_mesh(),
        compiler_params=_SC_PARAMS,
        scratch_types=[
            pltpu.VMEM_SHARED((NPAD, F), jnp.float32),
            pltpu.VMEM((BK, CH), jnp.int32),
            pltpu.VMEM((BK, CH), jnp.int32),
            pltpu.VMEM((BK, CH, F), jnp.float32),
            pltpu.VMEM((ZR, F), jnp.float32),
            pltpu.SemaphoreType.DMA,
            pltpu.SemaphoreType.DMA,
        ],
    )


def _pool_kernel():
    nchb = NPAD // CH
    wcr = nchb // NW

    def body(h_ref, b2d_ref, psum_ref, pcnt_ref,
             accp, accc, bidx, rows, ones, zp, zc, ssem):
        c = lax.axis_index("c")
        s = lax.axis_index("s")
        _fill(ones, CH, F, 1.0)

        @pl.when(s == 0)
        def _():
            _fill(zp, GP, 32, 0.0)
            _fill(zc, GP, F, 0.0)
            pltpu.sync_copy(zp, accp)
            pltpu.sync_copy(zc, accc)
        plsc.subcore_barrier()

        base = (c * NSUB + s) * wcr
        pltpu.sync_copy(b2d_ref.at[pl.ds(base, wcr)], bidx)

        def blk(b, _):
            pltpu.sync_copy(h_ref.at[pl.ds((base + b) * CH, CH)], rows)
            pltpu.sync_copy(rows, accp.at[bidx.at[b]], add=True)
            pltpu.async_copy(ones, accc.at[bidx.at[b]], ssem, add=True).wait()
            return 0
        lax.fori_loop(0, wcr, blk, 0)
        plsc.subcore_barrier()

        @pl.when(s == 0)
        def _():
            pltpu.sync_copy(accp, psum_ref.at[c])
            pltpu.sync_copy(accc, pcnt_ref.at[c])

    return pl.kernel(
        body,
        out_type=(jax.ShapeDtypeStruct((NCORE, GP, 32), jnp.float32),
                  jax.ShapeDtypeStruct((NCORE, GP, F), jnp.float32)),
        mesh=_mesh(),
        compiler_params=_SC_PARAMS,
        scratch_types=[
            pltpu.VMEM_SHARED((GP, 32), jnp.float32),
            pltpu.VMEM_SHARED((GP, F), jnp.float32),
            pltpu.VMEM((25, CH), jnp.int32),
            pltpu.VMEM((CH, 32), jnp.float32),
            pltpu.VMEM((CH, F), jnp.float32),
            pltpu.VMEM((GP, 32), jnp.float32),
            pltpu.VMEM((GP, F), jnp.float32),
            pltpu.SemaphoreType.DMA,
        ],
    )


RB = 2048
NBLK_TC = NPAD // RB


def _b1_body(degp, x, w1, g1, dinv):
    deg = degp[0, :, 0:1] + degp[1, :, 0:1] + 1.0
    di = lax.rsqrt(deg)
    h = jnp.dot(x[...], w1[...], preferred_element_type=jnp.float32)
    g1[...] = jnp.concatenate(
        [h * di, jnp.zeros((h.shape[0], 8), jnp.float32)], axis=1)
    dinv[...] = di


def _tc_b1(x_pad, degp, W1):
    return pl.pallas_call(
        _b1_body,
        grid=(NBLK_TC,),
        in_specs=[
            pl.BlockSpec((NCORE, RB, F), lambda i: (0, i, 0)),
            pl.BlockSpec((RB, 9), lambda i: (i, 0)),
            pl.BlockSpec((9, 8), lambda i: (0, 0)),
        ],
        out_specs=[
            pl.BlockSpec((RB, F), lambda i: (i, 0)),
            pl.BlockSpec((RB, 1), lambda i: (i, 0)),
        ],
        out_shape=[
            jax.ShapeDtypeStruct((NPAD, F), jnp.float32),
            jax.ShapeDtypeStruct((NPAD, 1), jnp.float32),
        ],
    )(degp, x_pad, W1)


def _mid_body(accp, g, dinv, w, b, gout):
    di = dinv[...]
    t = di * (accp[0, :, :8] + accp[1, :, :8] + g[:, :8]) + b[...]
    t = jnp.maximum(t, 0.0)
    h = jnp.dot(t, w[...], preferred_element_type=jnp.float32)
    gout[...] = h * di


def _tc_mid(accp, g, dinv, W, b):
    return pl.pallas_call(
        _mid_body,
        grid=(NBLK_TC,),
        in_specs=[
            pl.BlockSpec((NCORE, RB, F), lambda i: (0, i, 0)),
            pl.BlockSpec((RB, F), lambda i: (i, 0)),
            pl.BlockSpec((RB, 1), lambda i: (i, 0)),
            pl.BlockSpec((8, 16), lambda i: (0, 0)),
            pl.BlockSpec((1, 8), lambda i: (0, 0)),
        ],
        out_specs=pl.BlockSpec((RB, F), lambda i: (i, 0)),
        out_shape=jax.ShapeDtypeStruct((NPAD, F), jnp.float32),
    )(accp, g, dinv, W, b)


def _b3_body(accp, g, dinv, w, b, gout):
    di = dinv[...]
    t = di * (accp[0] + accp[1] + g[...]) + b[...]
    t = jnp.maximum(t, 0.0)
    h = jnp.dot(t, w[...], preferred_element_type=jnp.float32)
    h = h * di
    gout[0] = h[:, :16]
    gout[1] = h[:, 16:]


def _tc_b3(accp, g2, dinv, W3, b2):
    return pl.pallas_call(
        _b3_body,
        grid=(NBLK_TC,),
        in_specs=[
            pl.BlockSpec((NCORE, RB, 16), lambda i: (0, i, 0)),
            pl.BlockSpec((RB, 16), lambda i: (i, 0)),
            pl.BlockSpec((RB, 1), lambda i: (i, 0)),
            pl.BlockSpec((16, 32), lambda i: (0, 0)),
            pl.BlockSpec((1, 16), lambda i: (0, 0)),
        ],
        out_specs=pl.BlockSpec((NCORE, RB, 16), lambda i: (0, i, 0)),
        out_shape=jax.ShapeDtypeStruct((NCORE, NPAD, 16), jnp.float32),
    )(accp, g2, dinv, W3, b2)


def _b4_body(acc3, g3, dinv, b, out):
    di = dinv[...]
    lo = di * (acc3[0] + g3[0])
    hi = di * (acc3[1] + g3[1])
    out[...] = jnp.concatenate([lo, hi], axis=1) + b[...]


def _tc_b4(acc3, g3, dinv, b3):
    return pl.pallas_call(
        _b4_body,
        grid=(NBLK_TC,),
        in_specs=[
            pl.BlockSpec((NCORE, RB, 16), lambda i: (0, i, 0)),
            pl.BlockSpec((NCORE, RB, 16), lambda i: (0, i, 0)),
            pl.BlockSpec((RB, 1), lambda i: (i, 0)),
            pl.BlockSpec((1, 32), lambda i: (0, 0)),
        ],
        out_specs=pl.BlockSpec((RB, 32), lambda i: (i, 0)),
        out_shape=jax.ShapeDtypeStruct((NPAD, 32), jnp.float32),
    )(acc3, g3, dinv, b3)


def _head_body(psum, pcnt, wl, bl, out):
    pooled = psum[0] + psum[1]
    cnt = pcnt[0, :, 0:1] + pcnt[1, :, 0:1]
    mean = pooled / jnp.maximum(cnt, 1.0)
    r = jnp.dot(mean, wl[...], preferred_element_type=jnp.float32) + bl[...]
    out[...] = r[:G]


def _tc_head(psum, pcnt, Wl, bl):
    return pl.pallas_call(
        _head_body,
        out_shape=jax.ShapeDtypeStruct((G, 2), jnp.float32),
    )(psum, pcnt, Wl, bl)


def kernel(x, edge_index, batch, W1, b1, W2, b2, W3, b3, Wl, bl):
    E = edge_index.shape[1]
    wch = -(-E // (NW * CH * BK)) * BK
    ech = wch * NW
    e_pad = ech * CH

    src = jnp.concatenate(
        [edge_index[0], jnp.full((e_pad - E,), N, jnp.int32)]).reshape(ech, CH)
    dst = jnp.concatenate(
        [edge_index[1], jnp.full((e_pad - E,), N, jnp.int32)]).reshape(ech, CH)
    x_pad = jnp.concatenate(
        [x, jnp.zeros((NPAD - N, 9), jnp.float32)], axis=0)
    b2d = jnp.concatenate(
        [batch, jnp.full((NPAD - N,), G, jnp.int32)]).reshape(NPAD // CH, CH)

    degp = _deg_kernel(ech)(dst)
    g1, dinv = _tc_b1(x_pad, degp, W1)
    acc1 = _agg_kernel(True, ech)(g1, src, dst)
    g2 = _tc_mid(acc1, g1, dinv, W2, b1.reshape(1, 8))
    acc2 = _agg_kernel(True, ech)(g2, src, dst)
    g3 = _tc_b3(acc2, g2, dinv, W3, b2.reshape(1, 16))
    acc3 = _agg_kernel(False, ech)(g3, src, dst)
    out3 = _tc_b4(acc3, g3, dinv, b3.reshape(1, 32))
    psum, pcnt = _pool_kernel()(out3, b2d)
    return _tc_head(psum, pcnt, Wl, bl.reshape(1, 2))
